# Initial kernel scaffold; baseline (speedup 1.0000x reference)
import jax
import jax.numpy as jnp
from jax.experimental import pallas as pl


def kernel(x, logdet):
    raise NotImplementedError("write your pallas kernel here")


# speedup vs baseline: 1.9881x; 1.9881x over previous
import jax
import jax.numpy as jnp
from jax import lax
from jax.experimental import pallas as pl
from jax.experimental.pallas import tpu as pltpu
from jax.experimental.pallas import tpu_sc as plsc

_B, _C, _H, _W = 32, 384, 56, 56
_R = _H * _W
_G = 16
_NCHUNK = _C // _G
_NC = 2


def _sc_reverse_channels(x3):
    mesh = plsc.VectorSubcoreMesh(core_axis_name="c", subcore_axis_name="s")

    @pl.kernel(
        out_type=jax.ShapeDtypeStruct((_B, _C, _R), jnp.float32),
        mesh=mesh,
        scratch_types=[
            pltpu.VMEM((_G, _R), jnp.float32),
            pltpu.VMEM((_G, _R), jnp.float32),
            pltpu.SemaphoreType.DMA,
            pltpu.SemaphoreType.DMA,
            pltpu.SemaphoreType.DMA,
            pltpu.SemaphoreType.DMA,
        ],
    )
    def body(x_hbm, o_hbm, buf0, buf1, isem0, isem1, osem0, osem1):
        w = lax.axis_index("s") * _NC + lax.axis_index("c")

        def start_in(k, buf, sem):
            pltpu.async_copy(x_hbm.at[w, pl.ds(_C - (k + 1) * _G, _G)], buf, sem)

        def wait_in(buf, sem):
            pltpu.make_async_copy(x_hbm.at[w, pl.ds(0, _G)], buf, sem).wait()

        def fire_outs(k, buf, sem):
            for j in range(_G):
                pltpu.async_copy(buf.at[_G - 1 - j], o_hbm.at[w, k * _G + j], sem)

        def drain_outs(buf, sem):
            pltpu.make_async_copy(buf, x_hbm.at[w, pl.ds(0, _G)], sem).wait()

        start_in(0, buf0, isem0)
        start_in(1, buf1, isem1)

        @pl.loop(0, _NCHUNK, step=2)
        def _(k):
            wait_in(buf0, isem0)
            fire_outs(k, buf0, osem0)
            drain_outs(buf0, osem0)

            @pl.when(k + 2 < _NCHUNK)
            def _():
                start_in(k + 2, buf0, isem0)

            wait_in(buf1, isem1)
            fire_outs(k + 1, buf1, osem1)
            drain_outs(buf1, osem1)

            @pl.when(k + 3 < _NCHUNK)
            def _():
                start_in(k + 3, buf1, isem1)

    return body(x3)


def kernel(x, logdet):
    x3 = x.reshape(_B, _C, _R)
    out3 = _sc_reverse_channels(x3)
    return (out3.reshape(_B, _C, _H, _W), logdet)
